# baseline (speedup 1.0000x reference)
import functools

import jax
import jax.numpy as jnp
from jax import lax
from jax.experimental import pallas as pl
from jax.experimental.pallas import tpu as pltpu
from jax.experimental.pallas import tpu_sc as plsc

_NUM_CORES = 2
_NUM_SUBCORES = 16
_NW = _NUM_CORES * _NUM_SUBCORES
_D = 32
_LANES = 16
_TILE_B = 128
_NBUF = 2
_TW = 128


@functools.cache
def _make_gather(batch: int, hist: int):
    B = batch * hist
    bw = batch // _NW
    nbt = bw // _TILE_B
    ncg = _D // 8
    t1_len = bw * _D
    slab = _D * batch
    mesh = plsc.VectorSubcoreMesh(core_axis_name="c", subcore_axis_name="s")

    @functools.partial(
        pl.kernel,
        out_type=jax.ShapeDtypeStruct((B * _D,), jnp.float32),
        mesh=mesh,
        scratch_types=[
            pltpu.VMEM((hist, bw), jnp.int32),
            pltpu.VMEM((bw // 4, _TW), jnp.float32),
            pltpu.VMEM((bw // 4, _TW), jnp.float32),
            pltpu.VMEM((bw // 4, _TW), jnp.float32),
            pltpu.VMEM((bw // 4, _TW), jnp.float32),
            pltpu.VMEM((t1_len,), jnp.float32),
            pltpu.VMEM((t1_len,), jnp.float32),
            pltpu.SemaphoreType.DMA,
            pltpu.SemaphoreType.DMA,
            pltpu.SemaphoreType.DMA,
            pltpu.SemaphoreType.DMA,
            pltpu.SemaphoreType.DMA,
            pltpu.SemaphoreType.DMA,
            pltpu.SemaphoreType.DMA,
        ],
        compiler_params=pltpu.CompilerParams(
            use_tc_tiling_on_sc=False, needs_layout_passes=False
        ),
    )
    def gather_kernel(
        table_hbm, idx_hbm, out_hbm,
        idx_v, r0, r1, r2, r3, t0, t1,
        g0, g1, g2, g3, isem, s0, s1,
    ):
        wid = lax.axis_index("s") * _NUM_CORES + lax.axis_index("c")
        col0 = wid * bw
        rows = (r0, r1, r2, r3)
        gsem = (g0, g1, g2, g3)
        tiles = (t0, t1)
        ssem = (s0, s1)

        for h in range(hist):
            pltpu.async_copy(
                idx_hbm.at[pl.ds(h * batch + col0, bw)], idx_v.at[h], isem
            )
        for h in range(hist):
            pltpu.make_async_copy(
                idx_hbm.at[pl.ds(0, bw)], idx_v.at[0], isem
            ).wait()

        lane = lax.iota(jnp.int32, _LANES)
        cvec0 = lane * 128
        cvec1 = cvec0 + 2048
        hw = bw // 4

        def fire_gather(h, s, rb):
            pltpu.async_copy(
                table_hbm.at[idx_v.at[h].at[pl.ds(s * hw, hw)]], rows[rb], gsem[rb]
            )

        def wait_gather(rb):
            pltpu.make_async_copy(
                table_hbm.at[idx_v.at[0].at[pl.ds(0, hw)]], rows[rb], gsem[rb]
            ).wait()

        def transpose(rb, tb, s):
            src = rows[rb]
            dst = tiles[tb]

            def tr_body(i, carry):
                for j in range(8):
                    b = i * 8 + j
                    bg = s * hw + b
                    boff = (bg >> 7) * (ncg * 8 * 128) + (bg & 127)
                    v0 = src[b, pl.ds(0, _LANES)]
                    v1 = src[b, pl.ds(_LANES, _LANES)]
                    plsc.store_scatter(dst, [cvec0 + boff], v0)
                    plsc.store_scatter(dst, [cvec1 + boff], v1)
                return carry

            lax.fori_loop(0, hw // 8, tr_body, 0)

        def fire_stores(h, tb):
            for bt in range(nbt):
                for cg in range(ncg):
                    pltpu.async_copy(
                        tiles[tb].at[pl.ds((bt * ncg + cg) * 1024, 1024)],
                        out_hbm.at[
                            pl.ds(
                                h * slab + cg * (batch * 8)
                                + (wid * nbt + bt) * 1024,
                                1024,
                            )
                        ],
                        ssem[tb],
                    )

        def drain_stores(tb):
            for _ in range(nbt * ncg):
                pltpu.make_async_copy(
                    tiles[tb].at[pl.ds(0, 1024)],
                    out_hbm.at[pl.ds(0, 1024)],
                    ssem[tb],
                ).wait()

        for s in range(4):
            fire_gather(0, s, s)

        def step(h, s, tb, drain, fire):
            wait_gather(s)
            if s == 0 and drain:
                drain_stores(tb)
            transpose(s, tb, s)
            if s == 3:
                fire_stores(h, tb)
            if fire:
                fire_gather(h + 1, s, s)

        def pair(h0, drain0, drain1, fire0, fire1):
            for s in range(4):
                step(h0, s, 0, drain0, fire0)
            for s in range(4):
                step(h0 + 1, s, 1, drain1, fire1)

        pair(0, False, False, True, True)

        def body(i2, carry):
            pair(2 + i2 * 2, True, True, True, True)
            return carry

        n_main = (hist - 4) // 2
        lax.fori_loop(0, n_main, body, 0)

        pair(hist - 2, True, True, True, False)

        for tb in range(2):
            drain_stores(tb)

    return gather_kernel


def kernel(x, table):
    batch, hist = x.shape
    idx = x.T.reshape(batch * hist).astype(jnp.int32)
    tp = jnp.pad(table, ((0, 0), (0, _TW - _D)))
    flat = _make_gather(batch, hist)(tp, idx)
    a = flat.reshape(hist, _D // 8, batch // _TILE_B, 8, _TILE_B)
    return a.transpose(2, 4, 0, 1, 3).reshape(batch, hist, _D)
